# baseline (speedup 1.0000x reference)
import functools

import jax
import jax.numpy as jnp
from jax import lax
from jax.experimental import pallas as pl
from jax.experimental.pallas import tpu as pltpu
from jax.experimental.pallas import tpu_sc as plsc

D_MODEL = 1024
SCALE = 32.0
NC, NS = 2, 16
NW = NC * NS
LANES = 16
CH = 32
NBUF = 3
VECS = CH * D_MODEL // LANES


@functools.lru_cache(maxsize=None)
def _make_lookup(b_len: int, t_len: int, vocab: int):
    b_total = b_len * t_len
    b_per_w = b_total // NW
    nch = b_per_w // CH
    mesh = plsc.VectorSubcoreMesh(
        core_axis_name="c", subcore_axis_name="s",
        num_cores=NC, num_subcores=NS)

    @functools.partial(
        pl.kernel,
        out_type=jax.ShapeDtypeStruct((b_len * t_len, D_MODEL),
                                      jnp.float32),
        mesh=mesh,
        scratch_types=[
            pltpu.VMEM((b_per_w,), jnp.int32),
            [pltpu.VMEM((CH,), jnp.int32)] * NBUF,
            [pltpu.VMEM((CH, D_MODEL), jnp.float32)] * NBUF,
            [pltpu.SemaphoreType.DMA] * NBUF,
            [pltpu.SemaphoreType.DMA] * NBUF,
        ],
    )
    def lookup(x_hbm, tab_hbm, out_hbm, idx_v, idxr, bufs, gsems, osems):
        wid = lax.axis_index("s") * NC + lax.axis_index("c")
        base = wid * b_per_w
        pltpu.sync_copy(x_hbm.at[wid], idx_v)

        def stage_idx(g):
            for v in range(CH // LANES):
                idxr[g % NBUF][pl.ds(v * LANES, LANES)] = (
                    idx_v[pl.ds(g * CH + v * LANES, LANES)])

        look = NBUF - 1
        for g in range(min(look, nch)):
            stage_idx(g)
            pltpu.async_copy(tab_hbm.at[idxr[g % NBUF]], bufs[g % NBUF],
                             gsems[g % NBUF])
        for g in range(nch):
            b = g % NBUF
            buf = bufs[b]
            pltpu.make_async_copy(tab_hbm.at[idxr[b]], buf, gsems[b]).wait()
            nxt = g + look
            if nxt < nch:
                nb = nxt % NBUF
                if nxt >= NBUF:
                    pltpu.make_async_copy(
                        bufs[nb],
                        out_hbm.at[pl.ds(base + (nxt - NBUF) * CH, CH)],
                        osems[nb]).wait()
                stage_idx(nxt)
                pltpu.async_copy(tab_hbm.at[idxr[nb]], bufs[nb], gsems[nb])

            @plsc.parallel_loop(0, VECS, step=1, unroll=16)
            def _(i, buf=buf):
                sl = pl.ds((i & (D_MODEL // LANES - 1)) * LANES, LANES)
                r = i >> 6
                buf[r, sl] = buf[r, sl] * SCALE

            pltpu.async_copy(buf, out_hbm.at[pl.ds(base + g * CH, CH)],
                             osems[b])
        for g in range(max(nch - NBUF, 0), nch):
            b = g % NBUF
            pltpu.make_async_copy(bufs[b],
                                  out_hbm.at[pl.ds(base + g * CH, CH)],
                                  osems[b]).wait()

    return lookup


def kernel(x, token_emb):
    b, t = x.shape
    x2 = x.reshape(NW, (b * t) // NW).astype(jnp.int32)
    out = _make_lookup(b, t, token_emb.shape[0])(x2, token_emb)
    return out.reshape(b, t, D_MODEL)


# speedup vs baseline: 1.0038x; 1.0038x over previous
import functools

import jax
import jax.numpy as jnp
from jax import lax
from jax.experimental import pallas as pl
from jax.experimental.pallas import tpu as pltpu
from jax.experimental.pallas import tpu_sc as plsc

D_MODEL = 1024
SCALE = 32.0
NC, NS = 2, 16
NW = NC * NS
LANES = 16
CH = 32
NBUF = 3
VECS = CH * D_MODEL // LANES


@functools.lru_cache(maxsize=None)
def _make_lookup(b_len: int, t_len: int, vocab: int):
    b_total = b_len * t_len
    b_per_w = b_total // NW
    nch = b_per_w // CH
    mesh = plsc.VectorSubcoreMesh(
        core_axis_name="c", subcore_axis_name="s",
        num_cores=NC, num_subcores=NS)

    @functools.partial(
        pl.kernel,
        out_type=jax.ShapeDtypeStruct((b_len * t_len, D_MODEL),
                                      jnp.float32),
        mesh=mesh,
        scratch_types=[
            pltpu.VMEM((b_per_w,), jnp.int32),
            [pltpu.VMEM((CH,), jnp.int32)] * NBUF,
            [pltpu.VMEM((CH, D_MODEL), jnp.float32)] * NBUF,
            [pltpu.SemaphoreType.DMA] * NBUF,
            [pltpu.SemaphoreType.DMA] * NBUF,
        ],
    )
    def lookup(x_hbm, tab_hbm, out_hbm, idx_v, idxr, bufs, gsems, osems):
        wid = lax.axis_index("s") * NC + lax.axis_index("c")
        base = wid * b_per_w
        pltpu.sync_copy(x_hbm.at[wid], idx_v)

        def stage_idx(g):
            for v in range(CH // LANES):
                idxr[g % NBUF][pl.ds(v * LANES, LANES)] = (
                    idx_v[pl.ds(g * CH + v * LANES, LANES)])

        look = NBUF - 1
        for g in range(min(look, nch)):
            stage_idx(g)
            pltpu.async_copy(tab_hbm.at[idxr[g % NBUF]], bufs[g % NBUF],
                             gsems[g % NBUF])
        for g in range(nch):
            b = g % NBUF
            buf = bufs[b]
            pltpu.make_async_copy(tab_hbm.at[idxr[b]], buf, gsems[b]).wait()
            nxt = g + look
            if nxt < nch:
                nb = nxt % NBUF
                if nxt >= NBUF:
                    pltpu.make_async_copy(
                        bufs[nb],
                        out_hbm.at[pl.ds(base + (nxt - NBUF) * CH, CH)],
                        osems[nb]).wait()
                stage_idx(nxt)
                pltpu.async_copy(tab_hbm.at[idxr[nb]], bufs[nb], gsems[nb])

            @plsc.parallel_loop(0, VECS, step=1, unroll=8)
            def _(i, buf=buf):
                sl = pl.ds((i & (D_MODEL // LANES - 1)) * LANES, LANES)
                r = i >> 6
                buf[r, sl] = buf[r, sl] * SCALE

            pltpu.async_copy(buf, out_hbm.at[pl.ds(base + g * CH, CH)],
                             osems[b])
        for g in range(max(nch - NBUF, 0), nch):
            b = g % NBUF
            pltpu.make_async_copy(bufs[b],
                                  out_hbm.at[pl.ds(base + g * CH, CH)],
                                  osems[b]).wait()

    return lookup


def kernel(x, token_emb):
    b, t = x.shape
    x2 = x.reshape(NW, (b * t) // NW).astype(jnp.int32)
    out = _make_lookup(b, t, token_emb.shape[0])(x2, token_emb)
    return out.reshape(b, t, D_MODEL)
